# baseline (speedup 1.0000x reference)
import functools

import jax
import jax.numpy as jnp
from jax import lax
from jax.experimental import pallas as pl
from jax.experimental.pallas import tpu as pltpu
from jax.experimental.pallas import tpu_sc as plsc

NUM_FIELDS = 26
EMBED_DIM = 32
BATCH = 16384
VOCAB_ALL = 2600000
ROWS128 = (VOCAB_ALL * EMBED_DIM) // 128
N = BATCH * NUM_FIELDS
PERIOD = 208
NC, NS = 2, 16
NW = NC * NS
L = 16

A_CHUNK = 200
A_NCH = ROWS128 // A_CHUNK
A_BASE = A_NCH // NW
A_EXTRA = A_NCH - A_BASE * NW

PER_W = N // NW
GROUP = 128
G_PER_W = PER_W // GROUP
K = 8
ROUNDS = G_PER_W // K


def _compact_body(tab_hbm, t128_hbm, s1, s2, carry_unused=None):
    wid = lax.axis_index("s") * NC + lax.axis_index("c")
    n_mine = A_BASE + jnp.where(wid < A_EXTRA, 1, 0)
    c0 = wid * A_BASE + jnp.minimum(wid, A_EXTRA)

    def chunk_body(k, carry):
        r0 = pl.multiple_of((c0 + k) * A_CHUNK, 8)
        pltpu.sync_copy(tab_hbm.at[pl.ds(pl.multiple_of(r0 * 4, 8),
                                         A_CHUNK * 4), :], s1)

        def pack_body(rr, c):
            for q in range(4):
                for h in range(2):
                    s2[rr, pl.ds(q * EMBED_DIM + h * L, L)] = (
                        s1[rr * 4 + q, pl.ds(h * L, L)])
            return c

        lax.fori_loop(0, A_CHUNK, pack_body, 0)
        pltpu.sync_copy(s2, t128_hbm.at[pl.ds(r0, A_CHUNK), :])
        return carry

    lax.fori_loop(0, n_mine, chunk_body, 0)


def _gather_body(idx_hbm, offs_hbm, table_hbm, out_hbm, idx_v, pat_v, rows_v,
                 gsem):
    wid = lax.axis_index("s") * NC + lax.axis_index("c")
    base = wid * PER_W

    pltpu.sync_copy(offs_hbm, pat_v)
    pltpu.sync_copy(idx_hbm.at[pl.ds(base, PER_W)], idx_v)

    def add_body(o, carry):
        t0 = o * PERIOD
        for j in range(PERIOD // L):
            sl = pl.ds(t0 + L * j, L)
            idx_v[sl] = idx_v[sl] + pat_v[pl.ds(L * j, L)]
        return carry

    lax.fori_loop(0, PER_W // PERIOD, add_body, 0)

    def round_body(rd, carry):
        g0 = rd * K
        copies = []
        for s in range(K):
            src = table_hbm.at[idx_v.at[pl.ds((g0 + s) * GROUP, GROUP)]]
            copies.append(pltpu.async_copy(src, rows_v.at[s], gsem))
        for cp in copies:
            cp.wait()
        for s in range(K):
            dst = out_hbm.at[pl.ds(base + (g0 + s) * GROUP, GROUP)]
            pltpu.sync_copy(rows_v.at[s], dst)
        return carry

    lax.fori_loop(0, ROUNDS, round_body, 0)


def kernel(batch, offsets, table):
    flat_idx = batch.astype(jnp.int32).reshape(N)
    offs_tiled = jnp.tile(offsets.astype(jnp.int32), PERIOD // NUM_FIELDS)

    mesh = plsc.VectorSubcoreMesh(core_axis_name="c", subcore_axis_name="s")

    compact = functools.partial(
        pl.kernel,
        mesh=mesh,
        out_type=jax.ShapeDtypeStruct((ROWS128, 128), jnp.float32),
        scratch_types=[
            pltpu.VMEM((A_CHUNK * 4, EMBED_DIM), jnp.float32),
            pltpu.VMEM((A_CHUNK, 128), jnp.float32),
        ],
        compiler_params=pltpu.CompilerParams(use_tc_tiling_on_sc=True,
                                             needs_layout_passes=False),
    )(_compact_body)
    t128 = compact(table)
    table_lin = t128.reshape(VOCAB_ALL, EMBED_DIM)

    gather = functools.partial(
        pl.kernel,
        mesh=mesh,
        out_type=jax.ShapeDtypeStruct((N, EMBED_DIM), jnp.float32),
        scratch_types=[
            pltpu.VMEM((PER_W,), jnp.int32),
            pltpu.VMEM((PERIOD,), jnp.int32),
            pltpu.VMEM((K, GROUP, EMBED_DIM), jnp.float32),
            pltpu.SemaphoreType.DMA,
        ],
        compiler_params=pltpu.CompilerParams(use_tc_tiling_on_sc=False),
    )(_gather_body)
    out_flat = gather(flat_idx, offs_tiled, table_lin)
    return out_flat.reshape(BATCH, NUM_FIELDS, EMBED_DIM)


# speedup vs baseline: 1.4058x; 1.4058x over previous
import functools

import jax
import jax.numpy as jnp
from jax import lax
from jax.experimental import pallas as pl
from jax.experimental.pallas import tpu as pltpu
from jax.experimental.pallas import tpu_sc as plsc

NUM_FIELDS = 26
EMBED_DIM = 32
BATCH = 16384
N = BATCH * NUM_FIELDS
PERIOD = 208
NC, NS = 2, 16
NW = NC * NS
PER_W = N // NW
GROUP = 128
G_PER_W = PER_W // GROUP
K = 8
ROUNDS = G_PER_W // K


def _emb_body(idx_hbm, offs_hbm, table_hbm, out_hbm, idx_v, pat_v, rows_v, gsem):
    wid = lax.axis_index("s") * NC + lax.axis_index("c")
    base = wid * PER_W

    pltpu.sync_copy(offs_hbm, pat_v)
    pltpu.sync_copy(idx_hbm.at[pl.ds(base, PER_W)], idx_v)

    def add_body(o, carry):
        t0 = o * PERIOD
        for j in range(PERIOD // 16):
            sl = pl.ds(t0 + 16 * j, 16)
            idx_v[sl] = idx_v[sl] + pat_v[pl.ds(16 * j, 16)]
        return carry

    lax.fori_loop(0, PER_W // PERIOD, add_body, 0)

    def round_body(rd, carry):
        g0 = rd * K
        copies = []
        for s in range(K):
            src = table_hbm.at[idx_v.at[pl.ds((g0 + s) * GROUP, GROUP)]]
            copies.append(pltpu.async_copy(src, rows_v.at[s], gsem))
        for cp in copies:
            cp.wait()
        for s in range(K):
            dst = out_hbm.at[pl.ds(base + (g0 + s) * GROUP, GROUP)]
            pltpu.sync_copy(rows_v.at[s], dst)
        return carry

    lax.fori_loop(0, ROUNDS, round_body, 0)


def kernel(batch, offsets, table):
    flat_idx = batch.astype(jnp.int32).reshape(N)
    offs_tiled = jnp.tile(offsets.astype(jnp.int32), PERIOD // NUM_FIELDS)

    mesh = plsc.VectorSubcoreMesh(core_axis_name="c", subcore_axis_name="s")
    run = functools.partial(
        pl.kernel,
        mesh=mesh,
        out_type=jax.ShapeDtypeStruct((N, EMBED_DIM), jnp.float32),
        scratch_types=[
            pltpu.VMEM((PER_W,), jnp.int32),
            pltpu.VMEM((PERIOD,), jnp.int32),
            pltpu.VMEM((K, GROUP, EMBED_DIM), jnp.float32),
            pltpu.SemaphoreType.DMA,
        ],
        compiler_params=pltpu.CompilerParams(use_tc_tiling_on_sc=False),
    )(_emb_body)
    out_flat = run(flat_idx, offs_tiled, table)
    return out_flat.reshape(BATCH, NUM_FIELDS, EMBED_DIM)
